# baseline (speedup 1.0000x reference)
import jax
import jax.numpy as jnp
from jax import lax
from jax.experimental import pallas as pl
from jax.experimental.pallas import tpu as pltpu
from jax.experimental.pallas import tpu_sc as plsc

D_MODEL = 2048
N_ROWS = 4 * 2048
NUM_CORES = 2
NUM_SUBCORES = 16
NW = NUM_CORES * NUM_SUBCORES
B_PER_W = N_ROWS // NW
CH = 8
N_CHUNKS = B_PER_W // CH
NSLOT = 3


def _pe_add_body(x_hbm, t_hbm, pe_hbm, out_hbm, idx_v,
                 bo0, bo1, bo2, bp0, bp1, bp2,
                 sx0, sx1, sx2, sp0, sp1, sp2, so0, so1, so2):
    bo = (bo0, bo1, bo2)
    bp = (bp0, bp1, bp2)
    sx = (sx0, sx1, sx2)
    sp = (sp0, sp1, sp2)
    so = (so0, so1, so2)

    c = lax.axis_index("c")
    s = lax.axis_index("s")
    wid = s * NUM_CORES + c
    base = wid * B_PER_W
    pltpu.sync_copy(t_hbm.at[pl.ds(base, B_PER_W)], idx_v)

    def start_x(g, slot):
        pltpu.async_copy(x_hbm.at[pl.ds(base + g * CH, CH)],
                         bo[slot], sx[slot])

    def start_pe(g, slot):
        pltpu.async_copy(pe_hbm.at[idx_v.at[pl.ds(g * CH, CH)]],
                         bp[slot], sp[slot])

    def wait_store(slot):
        pltpu.make_async_copy(bo[slot], out_hbm.at[pl.ds(0, CH)],
                              so[slot]).wait()

    def do_chunk(g, slot, px, ppe, store_pending):
        if px is not None:
            if store_pending:
                wait_store((slot + 1) % NSLOT)
            start_x(px, (slot + 1) % NSLOT)
        if ppe is not None:
            start_pe(ppe, (slot + 2) % NSLOT)
        pltpu.make_async_copy(x_hbm.at[pl.ds(0, CH)], bo[slot],
                              sx[slot]).wait()
        pltpu.make_async_copy(pe_hbm.at[pl.ds(0, CH)], bp[slot],
                              sp[slot]).wait()

        def row_add(r, c2):
            for k in range(D_MODEL // 16):
                sl = pl.ds(k * 16, 16)
                bo[slot][r, sl] = bo[slot][r, sl] + bp[slot][r, sl]
            return c2

        lax.fori_loop(0, CH, row_add, 0)
        pltpu.async_copy(bo[slot], out_hbm.at[pl.ds(base + g * CH, CH)],
                         so[slot])

    start_x(0, 0)
    start_pe(0, 0)
    start_pe(1, 1)
    do_chunk(0, 0, 1, 2, False)
    do_chunk(1, 1, 2, 3, False)

    def group(gg, carry):
        g0 = 2 + gg * NSLOT
        do_chunk(g0 + 0, 2, g0 + 1, g0 + 2, True)
        do_chunk(g0 + 1, 0, g0 + 2, g0 + 3, True)
        do_chunk(g0 + 2, 1, g0 + 3, g0 + 4, True)
        return carry

    lax.fori_loop(0, 9, group, 0)

    do_chunk(29, 2, 30, 31, True)
    do_chunk(30, 0, 31, None, True)
    do_chunk(31, 1, None, None, False)

    for slot in range(NSLOT):
        wait_store(slot)


def kernel(x, t, pe):
    b, s, d = x.shape
    x2 = x.reshape(N_ROWS, D_MODEL)
    t1 = t.reshape(N_ROWS)

    mesh = plsc.VectorSubcoreMesh(
        core_axis_name="c",
        subcore_axis_name="s",
        num_cores=NUM_CORES,
        num_subcores=NUM_SUBCORES,
    )
    buf = pltpu.VMEM((CH, D_MODEL), jnp.float32)
    sem = pltpu.SemaphoreType.DMA
    run = pl.kernel(
        _pe_add_body,
        out_type=jax.ShapeDtypeStruct((N_ROWS, D_MODEL), jnp.float32),
        mesh=mesh,
        scratch_types=[
            pltpu.VMEM((B_PER_W,), jnp.int32),
            buf, buf, buf, buf, buf, buf,
            sem, sem, sem, sem, sem, sem, sem, sem, sem,
        ],
    )
    out = run(x2, t1, pe)
    return out.reshape(b, s, d)


# speedup vs baseline: 1.2654x; 1.2654x over previous
import jax
import jax.numpy as jnp
from jax import lax
from jax.experimental import pallas as pl
from jax.experimental.pallas import tpu as pltpu
from jax.experimental.pallas import tpu_sc as plsc

D_MODEL = 2048
N_ROWS = 4 * 2048
NUM_CORES = 2
NUM_SUBCORES = 16
NW = NUM_CORES * NUM_SUBCORES
B_PER_W = N_ROWS // NW
CH = 8
N_CHUNKS = B_PER_W // CH
GROUP = 6


def _pe_add_body(x_hbm, t_hbm, pe_hbm, out_hbm, idx_v,
                 bx0, bx1, bp0, bp1, bp2, bo0, bo1,
                 sx0, sx1, sp0, sp1, sp2, so0, so1):
    bx = (bx0, bx1)
    bp = (bp0, bp1, bp2)
    bo = (bo0, bo1)
    sx = (sx0, sx1)
    sp = (sp0, sp1, sp2)
    so = (so0, so1)

    c = lax.axis_index("c")
    s = lax.axis_index("s")
    wid = s * NUM_CORES + c
    base = wid * B_PER_W
    pltpu.sync_copy(t_hbm.at[pl.ds(base, B_PER_W)], idx_v)

    def start_x(g, slot):
        pltpu.async_copy(x_hbm.at[pl.ds(base + g * CH, CH)],
                         bx[slot], sx[slot])

    def start_pe(g, slot):
        pltpu.async_copy(pe_hbm.at[idx_v.at[pl.ds(g * CH, CH)]],
                         bp[slot], sp[slot])

    def wait_store(slot):
        pltpu.make_async_copy(bo[slot], out_hbm.at[pl.ds(0, CH)],
                              so[slot]).wait()

    def do_chunk(g, bslot, pslot, px, ppe, guard_store):
        if px is not None:
            start_x(px, (bslot + 1) % 2)
        if ppe is not None:
            start_pe(ppe, (pslot + 2) % 3)
        pltpu.make_async_copy(x_hbm.at[pl.ds(0, CH)], bx[bslot],
                              sx[bslot]).wait()
        pltpu.make_async_copy(pe_hbm.at[pl.ds(0, CH)], bp[pslot],
                              sp[pslot]).wait()
        if guard_store is None:
            wait_store(bslot)
        elif guard_store:
            @pl.when(g >= 2)
            def _():
                wait_store(bslot)

        def row_add(r, c2):
            for k in range(D_MODEL // 16):
                sl = pl.ds(k * 16, 16)
                bo[bslot][r, sl] = bx[bslot][r, sl] + bp[pslot][r, sl]
            return c2

        lax.fori_loop(0, CH, row_add, 0)
        pltpu.async_copy(bo[bslot], out_hbm.at[pl.ds(base + g * CH, CH)],
                         so[bslot])

    start_x(0, 0)
    start_pe(0, 0)
    start_pe(1, 1)

    def group(gg, carry):
        g0 = gg * GROUP
        for j in range(GROUP):
            do_chunk(g0 + j, j % 2, j % 3, g0 + j + 1, g0 + j + 2, True)
        return carry

    lax.fori_loop(0, 5, group, 0)

    do_chunk(30, 0, 0, 31, None, None)
    do_chunk(31, 1, 1, None, None, None)

    wait_store(0)
    wait_store(1)


def kernel(x, t, pe):
    b, s, d = x.shape
    x2 = x.reshape(N_ROWS, D_MODEL)
    t1 = t.reshape(N_ROWS)

    mesh = plsc.VectorSubcoreMesh(
        core_axis_name="c",
        subcore_axis_name="s",
        num_cores=NUM_CORES,
        num_subcores=NUM_SUBCORES,
    )
    buf = pltpu.VMEM((CH, D_MODEL), jnp.float32)
    sem = pltpu.SemaphoreType.DMA
    run = pl.kernel(
        _pe_add_body,
        out_type=jax.ShapeDtypeStruct((N_ROWS, D_MODEL), jnp.float32),
        mesh=mesh,
        scratch_types=[
            pltpu.VMEM((B_PER_W,), jnp.int32),
            buf, buf, buf, buf, buf, buf, buf,
            sem, sem, sem, sem, sem, sem, sem,
        ],
    )
    out = run(x2, t1, pe)
    return out.reshape(b, s, d)


# speedup vs baseline: 1.4360x; 1.1349x over previous
import jax
import jax.numpy as jnp
from jax import lax
from jax.experimental import pallas as pl
from jax.experimental.pallas import tpu as pltpu
from jax.experimental.pallas import tpu_sc as plsc

D_MODEL = 2048
N_ROWS = 4 * 2048
NUM_CORES = 2
NUM_SUBCORES = 16
NW = NUM_CORES * NUM_SUBCORES
B_PER_W = N_ROWS // NW
CH = 8
N_CHUNKS = B_PER_W // CH
NBUF = 2
N_GROUPS = N_CHUNKS // NBUF


def _pe_add_body(x_hbm, t_hbm, pe_hbm, out_hbm, idx_v,
                 bx0, bx1, bp0, bp1, bo0, bo1,
                 sx0, sx1, sp0, sp1, so0, so1):
    bx = (bx0, bx1)
    bp = (bp0, bp1)
    bo = (bo0, bo1)
    sx = (sx0, sx1)
    sp = (sp0, sp1)
    so = (so0, so1)

    c = lax.axis_index("c")
    s = lax.axis_index("s")
    wid = s * NUM_CORES + c
    base = wid * B_PER_W
    pltpu.sync_copy(t_hbm.at[pl.ds(base, B_PER_W)], idx_v)

    def start_loads(g, slot):
        row0 = base + g * CH
        pltpu.async_copy(x_hbm.at[pl.ds(row0, CH)], bx[slot], sx[slot])
        pltpu.async_copy(
            pe_hbm.at[idx_v.at[pl.ds(g * CH, CH)]], bp[slot], sp[slot])

    start_loads(0, 0)

    def group(gg, carry):
        for b in range(NBUF):
            g = gg * NBUF + b
            nb = 1 - b
            @pl.when(g + 1 < N_CHUNKS)
            def _():
                start_loads(g + 1, nb)

            pltpu.make_async_copy(
                x_hbm.at[pl.ds(0, CH)], bx[b], sx[b]).wait()
            pltpu.make_async_copy(
                pe_hbm.at[pl.ds(0, CH)], bp[b], sp[b]).wait()

            @pl.when(g >= NBUF)
            def _():
                pltpu.make_async_copy(
                    bo[b], out_hbm.at[pl.ds(0, CH)], so[b]).wait()

            def row_add(r, c2):
                for k in range(D_MODEL // 16):
                    sl = pl.ds(k * 16, 16)
                    bo[b][r, sl] = bx[b][r, sl] + bp[b][r, sl]
                return c2

            lax.fori_loop(0, CH, row_add, 0)

            row0 = base + g * CH
            pltpu.async_copy(bo[b], out_hbm.at[pl.ds(row0, CH)], so[b])
        return carry

    lax.fori_loop(0, N_GROUPS, group, 0)

    for b in range(NBUF):
        pltpu.make_async_copy(bo[b], out_hbm.at[pl.ds(0, CH)], so[b]).wait()


def kernel(x, t, pe):
    b, s, d = x.shape
    x2 = x.reshape(N_ROWS, D_MODEL)
    t1 = t.reshape(N_ROWS)

    mesh = plsc.VectorSubcoreMesh(
        core_axis_name="c",
        subcore_axis_name="s",
        num_cores=NUM_CORES,
        num_subcores=NUM_SUBCORES,
    )
    buf = pltpu.VMEM((CH, D_MODEL), jnp.float32)
    run = pl.kernel(
        _pe_add_body,
        out_type=jax.ShapeDtypeStruct((N_ROWS, D_MODEL), jnp.float32),
        mesh=mesh,
        scratch_types=[
            pltpu.VMEM((B_PER_W,), jnp.int32),
            buf, buf, buf, buf, buf, buf,
            pltpu.SemaphoreType.DMA, pltpu.SemaphoreType.DMA,
            pltpu.SemaphoreType.DMA, pltpu.SemaphoreType.DMA,
            pltpu.SemaphoreType.DMA, pltpu.SemaphoreType.DMA,
        ],
    )
    out = run(x2, t1, pe)
    return out.reshape(b, s, d)
